# baseline (speedup 1.0000x reference)
import functools

import numpy as np

import jax
import jax.numpy as jnp
from jax import lax
from jax.experimental import pallas as pl
from jax.experimental.pallas import tpu as pltpu
from jax.experimental.pallas import tpu_sc as plsc

N = 10000
E = 320000
D_FEAT = 128
H1 = 14
C = 7
LD = 7
DP = 16
NC = 2
NS = 16
NW = NC * NS
EPT = E // NW
CH = 80
NCH = EPT // CH
ACC_N = 10240
RPT = ACC_N // NS


def _edge_pass_body(W, h_hbm, src_hbm, dst_hbm, w_hbm, out_hbm,
                    src_a, dst_a, w_a, rows_a, sem_ai, sem_ag,
                    src_b, dst_b, w_b, rows_b, sem_bi, sem_bg,
                    zero_v, acc_sh):
    c = lax.axis_index("c")
    s = lax.axis_index("s")
    wid = c * NS + s

    def fire_idx(ci, srcb, dstb, wb, sem):
        base = wid * EPT + ci * CH
        pltpu.async_copy(src_hbm.at[pl.ds(base, CH)], srcb, sem)
        pltpu.async_copy(dst_hbm.at[pl.ds(base, CH)], dstb, sem)
        pltpu.async_copy(w_hbm.at[pl.ds(base, CH)], wb, sem)

    def wait_idx(srcb, dstb, wb, sem):
        pltpu.make_async_copy(src_hbm.at[pl.ds(0, CH)], srcb, sem).wait()
        pltpu.make_async_copy(dst_hbm.at[pl.ds(0, CH)], dstb, sem).wait()
        pltpu.make_async_copy(w_hbm.at[pl.ds(0, CH)], wb, sem).wait()

    def fire_gather(srcb, rowsb, sem):
        pltpu.async_copy(h_hbm.at[srcb], rowsb, sem)

    def wait_gather(srcb, rowsb, sem):
        pltpu.make_async_copy(h_hbm.at[srcb], rowsb, sem).wait()

    def process(rowsb, wb, dstb):
        for g in range(CH // 16):
            w16 = wb[pl.ds(g * 16, 16)]
            for j in range(16):
                e = g * 16 + j
                for h in range(W // 16):
                    rowsb[e, pl.ds(h * 16, 16)] = (
                        rowsb[e, pl.ds(h * 16, 16)] * w16[j])
        pltpu.sync_copy(rowsb, acc_sh.at[dstb], add=True)

    def zloop(i, carry):
        for h in range(W // 16):
            zero_v[i, pl.ds(h * 16, 16)] = jnp.zeros((16,), jnp.float32)
        return carry
    lax.fori_loop(0, RPT, zloop, 0)
    pltpu.sync_copy(zero_v, acc_sh.at[pl.ds(s * RPT, RPT)])
    plsc.subcore_barrier()

    fire_idx(0, src_a, dst_a, w_a, sem_ai)
    fire_idx(1, src_b, dst_b, w_b, sem_bi)
    wait_idx(src_a, dst_a, w_a, sem_ai)
    fire_gather(src_a, rows_a, sem_ag)

    def pair(k, carry):
        wait_idx(src_b, dst_b, w_b, sem_bi)
        fire_gather(src_b, rows_b, sem_bg)
        wait_gather(src_a, rows_a, sem_ag)
        process(rows_a, w_a, dst_a)
        fire_idx(2 * k + 2, src_a, dst_a, w_a, sem_ai)
        wait_gather(src_b, rows_b, sem_bg)
        process(rows_b, w_b, dst_b)

        @pl.when(2 * k + 3 < NCH)
        def _():
            fire_idx(2 * k + 3, src_b, dst_b, w_b, sem_bi)

        wait_idx(src_a, dst_a, w_a, sem_ai)
        fire_gather(src_a, rows_a, sem_ag)
        return carry
    lax.fori_loop(0, (NCH - 1) // 2, pair, 0)

    wait_gather(src_a, rows_a, sem_ag)
    process(rows_a, w_a, dst_a)

    plsc.subcore_barrier()
    pltpu.sync_copy(acc_sh.at[pl.ds(s * RPT, RPT)],
                    out_hbm.at[c, pl.ds(s * RPT, RPT)])


def _make_edge_pass(W):
    return pl.kernel(
        functools.partial(_edge_pass_body, W),
        out_type=jax.ShapeDtypeStruct((NC, ACC_N, W), jnp.float32),
        mesh=plsc.VectorSubcoreMesh(core_axis_name="c", subcore_axis_name="s"),
        scratch_types=[
            pltpu.VMEM((CH,), jnp.int32),
            pltpu.VMEM((CH,), jnp.int32),
            pltpu.VMEM((CH,), jnp.float32),
            pltpu.VMEM((CH, W), jnp.float32),
            pltpu.SemaphoreType.DMA,
            pltpu.SemaphoreType.DMA,
            pltpu.VMEM((CH,), jnp.int32),
            pltpu.VMEM((CH,), jnp.int32),
            pltpu.VMEM((CH,), jnp.float32),
            pltpu.VMEM((CH, W), jnp.float32),
            pltpu.SemaphoreType.DMA,
            pltpu.SemaphoreType.DMA,
            pltpu.VMEM((RPT, W), jnp.float32),
            pltpu.VMEM_SHARED((ACC_N, W), jnp.float32),
        ],
        compiler_params=pltpu.CompilerParams(use_tc_tiling_on_sc=False),
    )


_edge_pass16 = _make_edge_pass(DP)
_edge_pass32 = _make_edge_pass(2 * DP)


def _k1_body(x_ref, w_ref, b_ref, ei_ref, o_ref, src_ref, dst_ref):
    o_ref[...] = (jnp.dot(x_ref[...], w_ref[...],
                          preferred_element_type=jnp.float32) + b_ref[...])
    src_ref[...] = ei_ref[1]
    dst_ref[...] = ei_ref[0]


BR = 1024

_NOISE = np.zeros((ACC_N, C * LD), np.float32)
_NOISE[:N] = np.asarray(
    jax.random.normal(jax.random.key(42), (N, C, LD), dtype=jnp.float32)
).reshape(N, C * LD)

_TT = np.zeros((LD, C * LD), np.float32)
_TR = np.zeros((C, C * LD), np.float32)
_S = np.zeros((C * LD, C), np.float32)
for _i in range(C):
    for _j in range(LD):
        _TT[_j, LD * _i + _j] = 1.0
        _TR[_i, LD * _i + _j] = 1.0
        _S[LD * _i + _j, _i] = 1.0


def _k3_body(p_ref, w23_ref, o_ref):
    a = jnp.maximum(p_ref[0] + p_ref[1], 0.0)
    col = lax.broadcasted_iota(jnp.int32, (BR, DP), 1)
    latp = jnp.where(col < H1, a,
                     jnp.where(col == DP - 1, 1.0, 0.0))
    hm = jnp.dot(latp, w23_ref[...],
                 preferred_element_type=jnp.float32)
    o_ref[...] = jnp.concatenate([latp, hm], axis=1)


def _k5_body(p_ref, nz_ref, w4_ref, tt_ref, tr_ref, s_ref, o_ref):
    hi = jax.lax.Precision.HIGHEST
    agg = p_ref[0] + p_ref[1]
    a3 = agg[:, DP:DP + C]
    m = jnp.max(a3, axis=1, keepdims=True)
    ex = jnp.exp(a3 - m)
    alphas = ex / jnp.sum(ex, axis=1, keepdims=True)
    zstd = 1.0 + jnp.exp(agg[:, DP + C:DP + 2 * C])
    zm = jnp.dot(agg[:, 0:DP], w4_ref[...],
                 preferred_element_type=jnp.float32, precision=hi)
    za = jnp.dot(alphas, tt_ref[...],
                 preferred_element_type=jnp.float32, precision=hi)
    zr = jnp.dot(zstd, tr_ref[...],
                 preferred_element_type=jnp.float32, precision=hi)
    prod = (zm + nz_ref[...] * zr) * za
    o_ref[...] = jnp.dot(prod, s_ref[...],
                         preferred_element_type=jnp.float32,
                         precision=hi)


def kernel(x, edge_index, edge_weight, W1, b1, W2, b2, W3, b3, W4, b4):
    f32 = jnp.float32
    W1p = jnp.zeros((D_FEAT, DP), f32).at[:, :H1].set(W1)
    b1p = jnp.zeros((1, DP), f32).at[0, :H1].set(b1)
    W23 = jnp.concatenate([W2, W3], axis=1)
    b23 = jnp.concatenate([b2, b3])
    W23p = (jnp.zeros((DP, DP), f32).at[:H1, :2 * C].set(W23)
            .at[DP - 1, :2 * C].set(b23))
    W4p = (jnp.zeros((DP, C * LD), f32).at[:H1, :].set(W4)
           .at[DP - 1, :].set(b4))

    h1p, src, dst = pl.pallas_call(
        _k1_body,
        out_shape=[jax.ShapeDtypeStruct((N, DP), f32),
                   jax.ShapeDtypeStruct((E,), jnp.int32),
                   jax.ShapeDtypeStruct((E,), jnp.int32)])(
            x, W1p, b1p, edge_index)
    p1 = _edge_pass16(h1p, src, dst, edge_weight)
    lat32 = pl.pallas_call(
        _k3_body,
        grid=(ACC_N // BR,),
        in_specs=[pl.BlockSpec((NC, BR, DP), lambda i: (0, i, 0)),
                  pl.BlockSpec((DP, DP), lambda i: (0, 0))],
        out_specs=pl.BlockSpec((BR, 2 * DP), lambda i: (i, 0)),
        out_shape=jax.ShapeDtypeStruct((ACC_N, 2 * DP), f32))(p1, W23p)
    p2 = _edge_pass32(lat32, src, dst, edge_weight)
    out = pl.pallas_call(
        _k5_body,
        grid=(ACC_N // BR,),
        in_specs=[
            pl.BlockSpec((NC, BR, 2 * DP), lambda i: (0, i, 0)),
            pl.BlockSpec((BR, C * LD), lambda i: (i, 0)),
            pl.BlockSpec((DP, C * LD), lambda i: (0, 0)),
            pl.BlockSpec((LD, C * LD), lambda i: (0, 0)),
            pl.BlockSpec((C, C * LD), lambda i: (0, 0)),
            pl.BlockSpec((C * LD, C), lambda i: (0, 0)),
        ],
        out_specs=pl.BlockSpec((BR, C), lambda i: (i, 0)),
        out_shape=jax.ShapeDtypeStruct((ACC_N, C), f32))(
            p2, jnp.asarray(_NOISE), W4p,
            jnp.asarray(_TT), jnp.asarray(_TR), jnp.asarray(_S))
    return out[:N]


# speedup vs baseline: 1.0170x; 1.0170x over previous
import functools

import numpy as np

import jax
import jax.numpy as jnp
from jax import lax
from jax.experimental import pallas as pl
from jax.experimental.pallas import tpu as pltpu
from jax.experimental.pallas import tpu_sc as plsc

N = 10000
E = 320000
D_FEAT = 128
H1 = 14
C = 7
LD = 7
DP = 16
NC = 2
NS = 16
NW = NC * NS
EPT = E // NW
CH = 80
NCH = EPT // CH
ACC_N = 10240
RPT = ACC_N // NS


def _edge_pass_body(W, h_hbm, src_hbm, dst_hbm, w_hbm, out_hbm,
                    src_a, dst_a, w_a, rows_a, sem_ai, sem_ag,
                    src_b, dst_b, w_b, rows_b, sem_bi, sem_bg,
                    zero_v, acc_sh):
    c = lax.axis_index("c")
    s = lax.axis_index("s")
    wid = c * NS + s

    def fire_idx(ci, srcb, dstb, wb, sem):
        base = wid * EPT + ci * CH
        pltpu.async_copy(src_hbm.at[pl.ds(base, CH)], srcb, sem)
        pltpu.async_copy(dst_hbm.at[pl.ds(base, CH)], dstb, sem)
        pltpu.async_copy(w_hbm.at[pl.ds(base, CH)], wb, sem)

    def wait_idx(srcb, dstb, wb, sem):
        pltpu.make_async_copy(src_hbm.at[pl.ds(0, CH)], srcb, sem).wait()
        pltpu.make_async_copy(dst_hbm.at[pl.ds(0, CH)], dstb, sem).wait()
        pltpu.make_async_copy(w_hbm.at[pl.ds(0, CH)], wb, sem).wait()

    def fire_gather(srcb, rowsb, sem):
        pltpu.async_copy(h_hbm.at[srcb], rowsb, sem)

    def wait_gather(srcb, rowsb, sem):
        pltpu.make_async_copy(h_hbm.at[srcb], rowsb, sem).wait()

    def process(rowsb, wb, dstb):
        for g in range(CH // 16):
            w16 = wb[pl.ds(g * 16, 16)]
            for j in range(16):
                e = g * 16 + j
                for h in range(W // 16):
                    rowsb[e, pl.ds(h * 16, 16)] = (
                        rowsb[e, pl.ds(h * 16, 16)] * w16[j])
        pltpu.sync_copy(rowsb, acc_sh.at[dstb], add=True)

    def zloop(i, carry):
        for h in range(W // 16):
            zero_v[i, pl.ds(h * 16, 16)] = jnp.zeros((16,), jnp.float32)
        return carry
    lax.fori_loop(0, RPT, zloop, 0)
    pltpu.sync_copy(zero_v, acc_sh.at[pl.ds(s * RPT, RPT)])
    plsc.subcore_barrier()

    fire_idx(0, src_a, dst_a, w_a, sem_ai)
    fire_idx(1, src_b, dst_b, w_b, sem_bi)
    wait_idx(src_a, dst_a, w_a, sem_ai)
    fire_gather(src_a, rows_a, sem_ag)

    def pair(k, carry):
        wait_idx(src_b, dst_b, w_b, sem_bi)
        fire_gather(src_b, rows_b, sem_bg)
        wait_gather(src_a, rows_a, sem_ag)
        process(rows_a, w_a, dst_a)
        fire_idx(2 * k + 2, src_a, dst_a, w_a, sem_ai)
        wait_gather(src_b, rows_b, sem_bg)
        process(rows_b, w_b, dst_b)

        @pl.when(2 * k + 3 < NCH)
        def _():
            fire_idx(2 * k + 3, src_b, dst_b, w_b, sem_bi)

        wait_idx(src_a, dst_a, w_a, sem_ai)
        fire_gather(src_a, rows_a, sem_ag)
        return carry
    lax.fori_loop(0, (NCH - 1) // 2, pair, 0)

    wait_gather(src_a, rows_a, sem_ag)
    process(rows_a, w_a, dst_a)

    plsc.subcore_barrier()
    pltpu.sync_copy(acc_sh.at[pl.ds(s * RPT, RPT)],
                    out_hbm.at[c, pl.ds(s * RPT, RPT)])


def _make_edge_pass(W):
    return pl.kernel(
        functools.partial(_edge_pass_body, W),
        out_type=jax.ShapeDtypeStruct((NC, ACC_N, W), jnp.float32),
        mesh=plsc.VectorSubcoreMesh(core_axis_name="c", subcore_axis_name="s"),
        scratch_types=[
            pltpu.VMEM((CH,), jnp.int32),
            pltpu.VMEM((CH,), jnp.int32),
            pltpu.VMEM((CH,), jnp.float32),
            pltpu.VMEM((CH, W), jnp.float32),
            pltpu.SemaphoreType.DMA,
            pltpu.SemaphoreType.DMA,
            pltpu.VMEM((CH,), jnp.int32),
            pltpu.VMEM((CH,), jnp.int32),
            pltpu.VMEM((CH,), jnp.float32),
            pltpu.VMEM((CH, W), jnp.float32),
            pltpu.SemaphoreType.DMA,
            pltpu.SemaphoreType.DMA,
            pltpu.VMEM((RPT, W), jnp.float32),
            pltpu.VMEM_SHARED((ACC_N, W), jnp.float32),
        ],
        compiler_params=pltpu.CompilerParams(use_tc_tiling_on_sc=False),
    )


_edge_pass16 = _make_edge_pass(DP)
_edge_pass32 = _make_edge_pass(2 * DP)


def _k1_body(x_ref, w_ref, b_ref, ei_ref, o_ref, src_ref, dst_ref):
    o_ref[...] = (jnp.dot(x_ref[...], w_ref[...],
                          preferred_element_type=jnp.float32) + b_ref[...])
    src_ref[...] = ei_ref[1]
    dst_ref[...] = ei_ref[0]


BR = 2048

_NOISE = np.zeros((ACC_N, C * LD), np.float32)
_NOISE[:N] = np.asarray(
    jax.random.normal(jax.random.key(42), (N, C, LD), dtype=jnp.float32)
).reshape(N, C * LD)

_TT = np.zeros((LD, C * LD), np.float32)
_TR = np.zeros((C, C * LD), np.float32)
_S = np.zeros((C * LD, C), np.float32)
for _i in range(C):
    for _j in range(LD):
        _TT[_j, LD * _i + _j] = 1.0
        _TR[_i, LD * _i + _j] = 1.0
        _S[LD * _i + _j, _i] = 1.0


def _k3_body(p_ref, w23_ref, o_ref):
    a = jnp.maximum(p_ref[0] + p_ref[1], 0.0)
    col = lax.broadcasted_iota(jnp.int32, (BR, DP), 1)
    latp = jnp.where(col < H1, a,
                     jnp.where(col == DP - 1, 1.0, 0.0))
    hm = jnp.dot(latp, w23_ref[...],
                 preferred_element_type=jnp.float32)
    o_ref[...] = jnp.concatenate([latp, hm], axis=1)


def _k5_body(p_ref, nz_ref, w4_ref, tt_ref, tr_ref, s_ref, o_ref):
    hi = jax.lax.Precision.HIGHEST
    agg = p_ref[0] + p_ref[1]
    a3 = agg[:, DP:DP + C]
    m = jnp.max(a3, axis=1, keepdims=True)
    ex = jnp.exp(a3 - m)
    alphas = ex / jnp.sum(ex, axis=1, keepdims=True)
    zstd = 1.0 + jnp.exp(agg[:, DP + C:DP + 2 * C])
    zm = jnp.dot(agg[:, 0:DP], w4_ref[...],
                 preferred_element_type=jnp.float32, precision=hi)
    za = jnp.dot(alphas, tt_ref[...],
                 preferred_element_type=jnp.float32, precision=hi)
    zr = jnp.dot(zstd, tr_ref[...],
                 preferred_element_type=jnp.float32, precision=hi)
    prod = (zm + nz_ref[...] * zr) * za
    o_ref[...] = jnp.dot(prod, s_ref[...],
                         preferred_element_type=jnp.float32,
                         precision=hi)


def kernel(x, edge_index, edge_weight, W1, b1, W2, b2, W3, b3, W4, b4):
    f32 = jnp.float32
    W1p = jnp.zeros((D_FEAT, DP), f32).at[:, :H1].set(W1)
    b1p = jnp.zeros((1, DP), f32).at[0, :H1].set(b1)
    W23 = jnp.concatenate([W2, W3], axis=1)
    b23 = jnp.concatenate([b2, b3])
    W23p = (jnp.zeros((DP, DP), f32).at[:H1, :2 * C].set(W23)
            .at[DP - 1, :2 * C].set(b23))
    W4p = (jnp.zeros((DP, C * LD), f32).at[:H1, :].set(W4)
           .at[DP - 1, :].set(b4))

    h1p, src, dst = pl.pallas_call(
        _k1_body,
        out_shape=[jax.ShapeDtypeStruct((N, DP), f32),
                   jax.ShapeDtypeStruct((E,), jnp.int32),
                   jax.ShapeDtypeStruct((E,), jnp.int32)])(
            x, W1p, b1p, edge_index)
    p1 = _edge_pass16(h1p, src, dst, edge_weight)
    lat32 = pl.pallas_call(
        _k3_body,
        grid=(ACC_N // BR,),
        in_specs=[pl.BlockSpec((NC, BR, DP), lambda i: (0, i, 0)),
                  pl.BlockSpec((DP, DP), lambda i: (0, 0))],
        out_specs=pl.BlockSpec((BR, 2 * DP), lambda i: (i, 0)),
        out_shape=jax.ShapeDtypeStruct((ACC_N, 2 * DP), f32))(p1, W23p)
    p2 = _edge_pass32(lat32, src, dst, edge_weight)
    out = pl.pallas_call(
        _k5_body,
        grid=(ACC_N // BR,),
        in_specs=[
            pl.BlockSpec((NC, BR, 2 * DP), lambda i: (0, i, 0)),
            pl.BlockSpec((BR, C * LD), lambda i: (i, 0)),
            pl.BlockSpec((DP, C * LD), lambda i: (0, 0)),
            pl.BlockSpec((LD, C * LD), lambda i: (0, 0)),
            pl.BlockSpec((C, C * LD), lambda i: (0, 0)),
            pl.BlockSpec((C * LD, C), lambda i: (0, 0)),
        ],
        out_specs=pl.BlockSpec((BR, C), lambda i: (i, 0)),
        out_shape=jax.ShapeDtypeStruct((ACC_N, C), f32))(
            p2, jnp.asarray(_NOISE), W4p,
            jnp.asarray(_TT), jnp.asarray(_TR), jnp.asarray(_S))
    return out[:N]


# speedup vs baseline: 1.0727x; 1.0548x over previous
import functools

import numpy as np

import jax
import jax.numpy as jnp
from jax import lax
from jax.experimental import pallas as pl
from jax.experimental.pallas import tpu as pltpu
from jax.experimental.pallas import tpu_sc as plsc

N = 10000
E = 320000
D_FEAT = 128
H1 = 14
C = 7
LD = 7
DP = 16
NC = 2
NS = 16
NW = NC * NS
EPT = E // NW
CH = 80
NCH = EPT // CH
ACC_N = 10240
RPT = ACC_N // NS


def _edge_pass_body(W, h_hbm, src_hbm, dst_hbm, w_hbm, out_hbm,
                    src_a, dst_a, w_a, rows_a, sem_ai, sem_ag,
                    src_b, dst_b, w_b, rows_b, sem_bi, sem_bg,
                    dsts_a, dsts_b, sem_as, sem_bs,
                    zero_v, acc_sh):
    c = lax.axis_index("c")
    s = lax.axis_index("s")
    wid = c * NS + s

    def fire_idx(ci, srcb, dstb, wb, sem):
        base = wid * EPT + ci * CH
        pltpu.async_copy(src_hbm.at[pl.ds(base, CH)], srcb, sem)
        pltpu.async_copy(dst_hbm.at[pl.ds(base, CH)], dstb, sem)
        pltpu.async_copy(w_hbm.at[pl.ds(base, CH)], wb, sem)

    def wait_idx(srcb, dstb, wb, sem):
        pltpu.make_async_copy(src_hbm.at[pl.ds(0, CH)], srcb, sem).wait()
        pltpu.make_async_copy(dst_hbm.at[pl.ds(0, CH)], dstb, sem).wait()
        pltpu.make_async_copy(w_hbm.at[pl.ds(0, CH)], wb, sem).wait()

    def fire_gather(srcb, rowsb, sem):
        pltpu.async_copy(h_hbm.at[srcb], rowsb, sem)

    def wait_gather(srcb, rowsb, sem):
        pltpu.make_async_copy(h_hbm.at[srcb], rowsb, sem).wait()

    def process(rowsb, wb, dstb, dstsb, sem_s):
        for g in range(CH // 16):
            w16 = wb[pl.ds(g * 16, 16)]
            for j in range(16):
                e = g * 16 + j
                for h in range(W // 16):
                    rowsb[e, pl.ds(h * 16, 16)] = (
                        rowsb[e, pl.ds(h * 16, 16)] * w16[j])
        for g in range(CH // 16):
            dstsb[pl.ds(g * 16, 16)] = dstb[pl.ds(g * 16, 16)]
        pltpu.async_copy(rowsb, acc_sh.at[dstsb], sem_s, add=True)

    def wait_scatter(rowsb, dstsb, sem_s):
        pltpu.make_async_copy(rowsb, acc_sh.at[dstsb], sem_s).wait()

    def zloop(i, carry):
        for h in range(W // 16):
            zero_v[i, pl.ds(h * 16, 16)] = jnp.zeros((16,), jnp.float32)
        return carry
    lax.fori_loop(0, RPT, zloop, 0)
    pltpu.sync_copy(zero_v, acc_sh.at[pl.ds(s * RPT, RPT)])
    plsc.subcore_barrier()

    fire_idx(0, src_a, dst_a, w_a, sem_ai)
    fire_idx(1, src_b, dst_b, w_b, sem_bi)
    wait_idx(src_a, dst_a, w_a, sem_ai)
    fire_gather(src_a, rows_a, sem_ag)

    def pair(k, carry):
        @pl.when(k > 0)
        def _():
            wait_scatter(rows_b, dsts_b, sem_bs)
        wait_idx(src_b, dst_b, w_b, sem_bi)
        fire_gather(src_b, rows_b, sem_bg)
        wait_gather(src_a, rows_a, sem_ag)
        process(rows_a, w_a, dst_a, dsts_a, sem_as)
        fire_idx(2 * k + 2, src_a, dst_a, w_a, sem_ai)
        wait_gather(src_b, rows_b, sem_bg)
        process(rows_b, w_b, dst_b, dsts_b, sem_bs)

        @pl.when(2 * k + 3 < NCH)
        def _():
            fire_idx(2 * k + 3, src_b, dst_b, w_b, sem_bi)

        wait_scatter(rows_a, dsts_a, sem_as)
        wait_idx(src_a, dst_a, w_a, sem_ai)
        fire_gather(src_a, rows_a, sem_ag)
        return carry
    lax.fori_loop(0, (NCH - 1) // 2, pair, 0)

    wait_scatter(rows_b, dsts_b, sem_bs)
    wait_gather(src_a, rows_a, sem_ag)
    process(rows_a, w_a, dst_a, dsts_a, sem_as)
    wait_scatter(rows_a, dsts_a, sem_as)

    plsc.subcore_barrier()
    pltpu.sync_copy(acc_sh.at[pl.ds(s * RPT, RPT)],
                    out_hbm.at[c, pl.ds(s * RPT, RPT)])


def _make_edge_pass(W):
    return pl.kernel(
        functools.partial(_edge_pass_body, W),
        out_type=jax.ShapeDtypeStruct((NC, ACC_N, W), jnp.float32),
        mesh=plsc.VectorSubcoreMesh(core_axis_name="c", subcore_axis_name="s"),
        scratch_types=[
            pltpu.VMEM((CH,), jnp.int32),
            pltpu.VMEM((CH,), jnp.int32),
            pltpu.VMEM((CH,), jnp.float32),
            pltpu.VMEM((CH, W), jnp.float32),
            pltpu.SemaphoreType.DMA,
            pltpu.SemaphoreType.DMA,
            pltpu.VMEM((CH,), jnp.int32),
            pltpu.VMEM((CH,), jnp.int32),
            pltpu.VMEM((CH,), jnp.float32),
            pltpu.VMEM((CH, W), jnp.float32),
            pltpu.SemaphoreType.DMA,
            pltpu.SemaphoreType.DMA,
            pltpu.VMEM((CH,), jnp.int32),
            pltpu.VMEM((CH,), jnp.int32),
            pltpu.SemaphoreType.DMA,
            pltpu.SemaphoreType.DMA,
            pltpu.VMEM((RPT, W), jnp.float32),
            pltpu.VMEM_SHARED((ACC_N, W), jnp.float32),
        ],
        compiler_params=pltpu.CompilerParams(use_tc_tiling_on_sc=False),
    )


_edge_pass16 = _make_edge_pass(DP)
_edge_pass32 = _make_edge_pass(2 * DP)


def _k1_body(x_ref, w_ref, b_ref, ei_ref, o_ref, src_ref, dst_ref):
    o_ref[...] = (jnp.dot(x_ref[...], w_ref[...],
                          preferred_element_type=jnp.float32) + b_ref[...])
    src_ref[...] = ei_ref[1]
    dst_ref[...] = ei_ref[0]


BR = 2048

_NOISE = np.zeros((ACC_N, C * LD), np.float32)
_NOISE[:N] = np.asarray(
    jax.random.normal(jax.random.key(42), (N, C, LD), dtype=jnp.float32)
).reshape(N, C * LD)

_TT = np.zeros((LD, C * LD), np.float32)
_TR = np.zeros((C, C * LD), np.float32)
_S = np.zeros((C * LD, C), np.float32)
for _i in range(C):
    for _j in range(LD):
        _TT[_j, LD * _i + _j] = 1.0
        _TR[_i, LD * _i + _j] = 1.0
        _S[LD * _i + _j, _i] = 1.0


def _k3_body(p_ref, w23_ref, o_ref):
    a = jnp.maximum(p_ref[0] + p_ref[1], 0.0)
    col = lax.broadcasted_iota(jnp.int32, (BR, DP), 1)
    latp = jnp.where(col < H1, a,
                     jnp.where(col == DP - 1, 1.0, 0.0))
    hm = jnp.dot(latp, w23_ref[...],
                 preferred_element_type=jnp.float32)
    o_ref[...] = jnp.concatenate([latp, hm], axis=1)


def _k5_body(p_ref, nz_ref, w4_ref, tt_ref, tr_ref, s_ref, o_ref):
    hi = jax.lax.Precision.HIGHEST
    agg = p_ref[0] + p_ref[1]
    a3 = agg[:, DP:DP + C]
    m = jnp.max(a3, axis=1, keepdims=True)
    ex = jnp.exp(a3 - m)
    alphas = ex / jnp.sum(ex, axis=1, keepdims=True)
    zstd = 1.0 + jnp.exp(agg[:, DP + C:DP + 2 * C])
    zm = jnp.dot(agg[:, 0:DP], w4_ref[...],
                 preferred_element_type=jnp.float32, precision=hi)
    za = jnp.dot(alphas, tt_ref[...],
                 preferred_element_type=jnp.float32, precision=hi)
    zr = jnp.dot(zstd, tr_ref[...],
                 preferred_element_type=jnp.float32, precision=hi)
    prod = (zm + nz_ref[...] * zr) * za
    o_ref[...] = jnp.dot(prod, s_ref[...],
                         preferred_element_type=jnp.float32,
                         precision=hi)


def kernel(x, edge_index, edge_weight, W1, b1, W2, b2, W3, b3, W4, b4):
    f32 = jnp.float32
    W1p = jnp.zeros((D_FEAT, DP), f32).at[:, :H1].set(W1)
    b1p = jnp.zeros((1, DP), f32).at[0, :H1].set(b1)
    W23 = jnp.concatenate([W2, W3], axis=1)
    b23 = jnp.concatenate([b2, b3])
    W23p = (jnp.zeros((DP, DP), f32).at[:H1, :2 * C].set(W23)
            .at[DP - 1, :2 * C].set(b23))
    W4p = (jnp.zeros((DP, C * LD), f32).at[:H1, :].set(W4)
           .at[DP - 1, :].set(b4))

    h1p, src, dst = pl.pallas_call(
        _k1_body,
        out_shape=[jax.ShapeDtypeStruct((N, DP), f32),
                   jax.ShapeDtypeStruct((E,), jnp.int32),
                   jax.ShapeDtypeStruct((E,), jnp.int32)])(
            x, W1p, b1p, edge_index)
    p1 = _edge_pass16(h1p, src, dst, edge_weight)
    lat32 = pl.pallas_call(
        _k3_body,
        grid=(ACC_N // BR,),
        in_specs=[pl.BlockSpec((NC, BR, DP), lambda i: (0, i, 0)),
                  pl.BlockSpec((DP, DP), lambda i: (0, 0))],
        out_specs=pl.BlockSpec((BR, 2 * DP), lambda i: (i, 0)),
        out_shape=jax.ShapeDtypeStruct((ACC_N, 2 * DP), f32))(p1, W23p)
    p2 = _edge_pass32(lat32, src, dst, edge_weight)
    out = pl.pallas_call(
        _k5_body,
        grid=(ACC_N // BR,),
        in_specs=[
            pl.BlockSpec((NC, BR, 2 * DP), lambda i: (0, i, 0)),
            pl.BlockSpec((BR, C * LD), lambda i: (i, 0)),
            pl.BlockSpec((DP, C * LD), lambda i: (0, 0)),
            pl.BlockSpec((LD, C * LD), lambda i: (0, 0)),
            pl.BlockSpec((C, C * LD), lambda i: (0, 0)),
            pl.BlockSpec((C * LD, C), lambda i: (0, 0)),
        ],
        out_specs=pl.BlockSpec((BR, C), lambda i: (i, 0)),
        out_shape=jax.ShapeDtypeStruct((ACC_N, C), f32))(
            p2, jnp.asarray(_NOISE), W4p,
            jnp.asarray(_TT), jnp.asarray(_TR), jnp.asarray(_S))
    return out[:N]
